# baseline (speedup 1.0000x reference)
import functools

import jax
import jax.numpy as jnp
from jax import lax
from jax.experimental import pallas as pl
from jax.experimental.pallas import tpu as pltpu
from jax.experimental.pallas import tpu_sc as plsc

N = 10000
E = 320000
D = 128
G = 64
MIN_SCORE_C = 0.05

NC = 2
NS = 16
NW = NC * NS
E_PER_W = E // NW
K = 80
N_CHUNKS = E_PER_W // K
NROW_CHUNKS = N // K
NBUF = 4
NROUND = -(-N_CHUNKS // NBUF)


@functools.cache
def _make_seg_sum(d_feat, d_acc):
    mesh = plsc.VectorSubcoreMesh(
        core_axis_name="c", subcore_axis_name="s", num_cores=NC, num_subcores=NS
    )

    @functools.partial(
        pl.kernel,
        mesh=mesh,
        out_type=jax.ShapeDtypeStruct((NC, N, d_acc), jnp.float32),
        scratch_types=[
            pltpu.VMEM((NBUF, 2, K), jnp.int32),
            pltpu.VMEM((NBUF, K, d_feat), jnp.float32),
            pltpu.VMEM_SHARED((N, d_acc), jnp.float32),
        ]
        + [pltpu.SemaphoreType.DMA] * (3 * NBUF),
    )
    def seg_sum(x_hbm, ei_hbm, out_hbm, idx, rows, acc, *sems):
        isem = sems[:NBUF]
        gsem = sems[NBUF : 2 * NBUF]
        ssem = sems[2 * NBUF :]
        cid = lax.axis_index("c")
        sid = lax.axis_index("s")
        wid = sid * NC + cid

        def idx_copy(b, c, make_only=False):
            f = pltpu.make_async_copy if make_only else pltpu.async_copy
            return (f(ei_hbm.at[wid, c], idx.at[b], isem[b]),)

        def gather_copy(b, c, make_only=False):
            f = pltpu.make_async_copy if make_only else pltpu.async_copy
            return f(x_hbm.at[idx.at[b, 0]], rows.at[b], gsem[b])

        def scatter_copy(b, c):
            pltpu.async_copy(
                rows.at[b], acc.at[idx.at[b, 1]], ssem[b], add=True
            )

        def scatter_wait(b):
            pltpu.make_async_copy(
                rows.at[b], acc.at[pl.ds(0, K)], ssem[b]
            ).wait()

        for b in range(NBUF):
            idx_copy(b, b)

        @pl.loop(0, K)
        def _zrow(i):
            @pl.loop(0, d_feat // 16)
            def _zlane(j):
                rows[0, i, pl.ds(j * 16, 16)] = jnp.zeros((16,), jnp.float32)

        @pl.loop(sid, NROW_CHUNKS, step=NS)
        def _zfire(cpy):
            pltpu.async_copy(rows.at[0], acc.at[pl.ds(cpy * K, K)], ssem[0])

        @pl.loop(sid, NROW_CHUNKS, step=NS)
        def _zwait(cpy):
            pltpu.make_async_copy(rows.at[0], acc.at[pl.ds(cpy * K, K)], ssem[0]).wait()

        plsc.subcore_barrier()

        for b in range(NBUF):
            for d in idx_copy(b, b, make_only=True):
                d.wait()
            gather_copy(b, b)

        @pl.loop(0, 30)
        def _round(r):
            c0 = r * NBUF
            for b in range(NBUF):
                gather_copy(b, c0 + b, make_only=True).wait()
                scatter_copy(b, c0 + b)
            for b in range(NBUF):
                scatter_wait(b)
                idx_copy(b, c0 + NBUF + b)
            for b in range(NBUF):
                for d in idx_copy(b, 0, make_only=True):
                    d.wait()
                gather_copy(b, c0 + NBUF + b)

        for b in range(NBUF):
            gather_copy(b, 120 + b, make_only=True).wait()
            scatter_copy(b, 120 + b)
        for b in range(NBUF):
            scatter_wait(b)
            if b == 0:
                idx_copy(0, 124)
        for d in idx_copy(0, 124, make_only=True):
            d.wait()
        gather_copy(0, 124)
        gather_copy(0, 124, make_only=True).wait()
        scatter_copy(0, 124)
        scatter_wait(0)

        plsc.subcore_barrier()

        MAXD = -(-NROW_CHUNKS // NS)
        for k in range(MAXD):
            b = k % NBUF
            cpy = sid + k * NS

            @pl.when(cpy < NROW_CHUNKS)
            def _dr():
                if k >= NBUF:
                    pltpu.make_async_copy(
                        rows.at[b], out_hbm.at[cid].at[pl.ds(0, K)], gsem[b]
                    ).wait()
                pltpu.async_copy(acc.at[pl.ds(cpy * K, K)], rows.at[b], isem[b])
                pltpu.make_async_copy(
                    acc.at[pl.ds(cpy * K, K)], rows.at[b], isem[b]
                ).wait()
                pltpu.async_copy(rows.at[b], out_hbm.at[cid].at[pl.ds(cpy * K, K)], gsem[b])

        for k in range(MAXD):
            b = k % NBUF
            cpy = sid + k * NS
            cpn = sid + (k + NBUF) * NS

            @pl.when((cpy < NROW_CHUNKS) & (cpn >= NROW_CHUNKS))
            def _drw():
                pltpu.make_async_copy(
                    rows.at[b], out_hbm.at[cid].at[pl.ds(0, K)], gsem[b]
                ).wait()

    return seg_sum


def _seg_sum_128(x, ei):
    return _make_seg_sum(D, D)(x, ei)


def _score_body(x_ref, p_ref, batch_ref, att_ref,
                score_ref, maskf_ref, attn_ref, ratio_ref):
    xb = x_ref[...]
    raw = jnp.sum(xb * p_ref[...], axis=1, keepdims=True)
    b = batch_ref[...]
    gi = lax.broadcasted_iota(jnp.int32, (N, G), 1)
    M = b == gi
    Mf = M.astype(jnp.float32)
    NEG = jnp.float32(-1e30)

    smax_g = jnp.max(jnp.where(M, raw, NEG), axis=0, keepdims=True)
    smax_g = jnp.where(smax_g > jnp.float32(-1e29), smax_g, 0.0)
    smax_node = jnp.sum(Mf * smax_g, axis=1, keepdims=True)
    ex = jnp.exp(raw - smax_node)
    ssum_g = jnp.sum(Mf * ex, axis=0, keepdims=True)
    ssum_node = jnp.sum(Mf * ssum_g, axis=1, keepdims=True)
    score = ex / jnp.maximum(ssum_node, 1e-16)

    scmax_g = jnp.max(jnp.where(M, score, NEG), axis=0, keepdims=True)
    scmax_g = jnp.where(scmax_g > jnp.float32(-1e29), scmax_g, 0.0)
    scmax_node = jnp.sum(Mf * scmax_g, axis=1, keepdims=True)
    thresh = jnp.minimum(scmax_node - 1e-7, MIN_SCORE_C)
    mask = score > thresh
    maskf = mask.astype(jnp.float32)

    n2 = jnp.sum(maskf)
    ratio_ref[...] = jnp.reshape(n2 * jnp.float32(1.0 / N), (1, 1))

    tgt = att_ref[...]
    kl = tgt * (jnp.log(jnp.maximum(tgt, 1e-30)) - jnp.log(score + 1e-14))
    kl = jnp.where(mask & (tgt > 0), kl, 0.0)
    counts = jnp.sum(Mf * maskf, axis=0, keepdims=True)
    attn_ref[...] = jnp.sum(Mf * kl, axis=0, keepdims=True) / jnp.maximum(counts, 1.0)

    score_ref[...] = score
    maskf_ref[...] = maskf


def _score_call(x, p, batch2, att2):
    return pl.pallas_call(
        _score_body,
        out_shape=[
            jax.ShapeDtypeStruct((N, 1), jnp.float32),
            jax.ShapeDtypeStruct((N, 1), jnp.float32),
            jax.ShapeDtypeStruct((1, G), jnp.float32),
            jax.ShapeDtypeStruct((1, 1), jnp.float32),
        ],
    )(x, p, batch2, att2)


RB = 2000
NBLK = N // RB


def _mlp1_body(a0_ref, a1_ref, x_ref, w1_ref, b1_ref, w2_ref, b2_ref,
               score_ref, maskf_ref, out_ref):
    inp = a0_ref[0] + a1_ref[0] + x_ref[...]
    t = jnp.dot(inp, w1_ref[...], preferred_element_type=jnp.float32) + b1_ref[...]
    t = jnp.maximum(t, 0.0)
    h = jnp.dot(t, w2_ref[...], preferred_element_type=jnp.float32) + b2_ref[...]
    h = jnp.maximum(h, 0.0)
    hx0 = h * score_ref[...] * maskf_ref[...]
    out_ref[...] = jnp.concatenate([hx0, jnp.zeros_like(hx0)], axis=1)


def _mlp1_call(agg, x, W1, b1, W2, b2, score, maskf):
    blk = lambda r, c: pl.BlockSpec((r, c), lambda i: (i, 0))
    full = lambda r, c: pl.BlockSpec((r, c), lambda i: (0, 0))
    part = lambda s: pl.BlockSpec((1, RB, D), lambda i, s=s: (s, i, 0))
    return pl.pallas_call(
        _mlp1_body,
        grid=(NBLK,),
        in_specs=[
            part(0), part(1), blk(RB, D),
            full(D, 256), full(1, 256), full(256, 64), full(1, 64),
            blk(RB, 1), blk(RB, 1),
        ],
        out_specs=blk(RB, 2 * 64),
        out_shape=jax.ShapeDtypeStruct((N, 2 * 64), jnp.float32),
    )(agg, agg, x, W1, b1.reshape(1, 256), W2, b2.reshape(1, 64), score, maskf)


def _mlp2_body(a0_ref, a1_ref, hx0_ref, w3_ref, b3_ref, w4_ref, b4_ref,
               w5_ref, b5_ref, maskf_ref, batch_ref, out_ref, g_scr):
    i = pl.program_id(0)
    inp = a0_ref[0] + a1_ref[0] + hx0_ref[...]
    t = jnp.dot(inp, w3_ref[...], preferred_element_type=jnp.float32) + b3_ref[...]
    t = jnp.maximum(t, 0.0)
    h2 = jnp.dot(t, w4_ref[...], preferred_element_type=jnp.float32) + b4_ref[...]
    h2 = jnp.maximum(h2, 0.0)
    h2m = h2 * maskf_ref[...]
    b = batch_ref[...]
    gi = lax.broadcasted_iota(jnp.int32, (RB, G), 1)
    Mf = (b == gi).astype(jnp.float32)
    partial = lax.dot_general(Mf, h2m, (((0,), (0,)), ((), ())),
                              preferred_element_type=jnp.float32)

    @pl.when(i == 0)
    def _():
        g_scr[...] = partial

    @pl.when(i > 0)
    def _():
        g_scr[...] = g_scr[...] + partial

    @pl.when(i == NBLK - 1)
    def _():
        out_ref[...] = (
            jnp.dot(g_scr[...], w5_ref[...], preferred_element_type=jnp.float32)
            + b5_ref[...]
        )


def _mlp2_call(agg, hx0, W3, b3, W4, b4, W5, b5, maskf, batch2):
    blk = lambda r, c: pl.BlockSpec((r, c), lambda i: (i, 0))
    full = lambda r, c: pl.BlockSpec((r, c), lambda i: (0, 0))
    part = lambda s: pl.BlockSpec((1, RB, D), lambda i, s=s: (s, i, 0))
    return pl.pallas_call(
        _mlp2_body,
        grid=(NBLK,),
        in_specs=[
            part(0), part(1), blk(RB, D),
            full(D, 256), full(1, 256), full(256, 64), full(1, 64),
            full(64, 1), full(1, 1),
            blk(RB, 1), blk(RB, 1),
        ],
        out_specs=full(G, 1),
        out_shape=jax.ShapeDtypeStruct((G, 1), jnp.float32),
        scratch_shapes=[pltpu.VMEM((G, 64), jnp.float32)],
    )(agg, agg, hx0, W3, b3.reshape(1, 256), W4, b4.reshape(1, 64),
      W5, b5.reshape(1, 1), maskf, batch2)


def kernel(x, edge_index, batch, node_attention, W1, b1, W2, b2, p, W3, b3, W4, b4, W5, b5):
    batch2 = batch.reshape(N, 1)
    att2 = node_attention.reshape(N, 1)

    ei4 = jnp.transpose(edge_index.reshape(2, NW, N_CHUNKS, K), (1, 2, 0, 3))
    agg1 = _seg_sum_128(x, ei4)

    score, maskf, attn, ratio = _score_call(x, p.reshape(1, D), batch2, att2)

    hx0 = _mlp1_call(agg1, x, W1, b1, W2, b2, score, maskf)

    agg2 = _seg_sum_128(hx0, ei4)

    W3p = jnp.concatenate([W3, jnp.zeros((64, 256), jnp.float32)], axis=0)
    out = _mlp2_call(agg2, hx0, W3p, b3, W4, b4, W5, b5, maskf, batch2)

    attn_loss = attn.reshape(G)
    ratio_s = ratio[0, 0]
    return out, attn_loss, ratio_s


# speedup vs baseline: 1.0156x; 1.0156x over previous
import functools

import jax
import jax.numpy as jnp
from jax import lax
from jax.experimental import pallas as pl
from jax.experimental.pallas import tpu as pltpu
from jax.experimental.pallas import tpu_sc as plsc

N = 10000
E = 320000
D = 128
G = 64
MIN_SCORE_C = 0.05

NC = 2
NS = 16
NW = NC * NS
E_PER_W = E // NW
K = 80
N_CHUNKS = E_PER_W // K
NROW_CHUNKS = N // K
NBUF = 4
NROUND = -(-N_CHUNKS // NBUF)


@functools.cache
def _make_seg_sum(d_feat, d_acc):
    mesh = plsc.VectorSubcoreMesh(
        core_axis_name="c", subcore_axis_name="s", num_cores=NC, num_subcores=NS
    )

    @functools.partial(
        pl.kernel,
        mesh=mesh,
        out_type=jax.ShapeDtypeStruct((NC, N, d_acc), jnp.float32),
        scratch_types=[
            pltpu.VMEM((NBUF, 2, K), jnp.int32),
            pltpu.VMEM((NBUF, K, d_feat), jnp.float32),
            pltpu.VMEM_SHARED((N, d_acc), jnp.float32),
        ]
        + [pltpu.SemaphoreType.DMA] * (3 * NBUF),
    )
    def seg_sum(x_hbm, ei_hbm, out_hbm, idx, rows, acc, *sems):
        isem = sems[:NBUF]
        gsem = sems[NBUF : 2 * NBUF]
        ssem = sems[2 * NBUF :]
        cid = lax.axis_index("c")
        sid = lax.axis_index("s")
        wid = sid * NC + cid

        def idx_copy(b, c, make_only=False):
            f = pltpu.make_async_copy if make_only else pltpu.async_copy
            off = wid * E_PER_W + c * K
            d0 = f(ei_hbm.at[pl.ds(off, K)], idx.at[b, 0], isem[b])
            d1 = f(ei_hbm.at[pl.ds(E + off, K)], idx.at[b, 1], isem[b])
            return (d0, d1)

        def gather_copy(b, c, make_only=False):
            f = pltpu.make_async_copy if make_only else pltpu.async_copy
            return f(x_hbm.at[idx.at[b, 0]], rows.at[b], gsem[b])

        def scatter_copy(b, c):
            pltpu.async_copy(
                rows.at[b], acc.at[idx.at[b, 1]], ssem[b], add=True
            )

        def scatter_wait(b):
            pltpu.make_async_copy(
                rows.at[b], acc.at[pl.ds(0, K)], ssem[b]
            ).wait()

        for b in range(NBUF):
            idx_copy(b, b)

        @pl.loop(0, K)
        def _zrow(i):
            @pl.loop(0, d_feat // 16)
            def _zlane(j):
                rows[0, i, pl.ds(j * 16, 16)] = jnp.zeros((16,), jnp.float32)

        @pl.loop(sid, NROW_CHUNKS, step=NS)
        def _zfire(cpy):
            pltpu.async_copy(rows.at[0], acc.at[pl.ds(cpy * K, K)], ssem[0])

        @pl.loop(sid, NROW_CHUNKS, step=NS)
        def _zwait(cpy):
            pltpu.make_async_copy(rows.at[0], acc.at[pl.ds(cpy * K, K)], ssem[0]).wait()

        plsc.subcore_barrier()

        for b in range(NBUF):
            for d in idx_copy(b, b, make_only=True):
                d.wait()
            gather_copy(b, b)

        @pl.loop(0, 30)
        def _round(r):
            c0 = r * NBUF
            for b in range(NBUF):
                gather_copy(b, c0 + b, make_only=True).wait()
                scatter_copy(b, c0 + b)
            for b in range(NBUF):
                scatter_wait(b)
                idx_copy(b, c0 + NBUF + b)
            for b in range(NBUF):
                for d in idx_copy(b, 0, make_only=True):
                    d.wait()
                gather_copy(b, c0 + NBUF + b)

        for b in range(NBUF):
            gather_copy(b, 120 + b, make_only=True).wait()
            scatter_copy(b, 120 + b)
        for b in range(NBUF):
            scatter_wait(b)
            if b == 0:
                idx_copy(0, 124)
        for d in idx_copy(0, 124, make_only=True):
            d.wait()
        gather_copy(0, 124)
        gather_copy(0, 124, make_only=True).wait()
        scatter_copy(0, 124)
        scatter_wait(0)

        plsc.subcore_barrier()

        MAXD = -(-NROW_CHUNKS // NS)
        for k in range(MAXD):
            b = k % NBUF
            cpy = sid + k * NS

            @pl.when(cpy < NROW_CHUNKS)
            def _dr():
                if k >= NBUF:
                    pltpu.make_async_copy(
                        rows.at[b], out_hbm.at[cid].at[pl.ds(0, K)], gsem[b]
                    ).wait()
                pltpu.async_copy(acc.at[pl.ds(cpy * K, K)], rows.at[b], isem[b])
                pltpu.make_async_copy(
                    acc.at[pl.ds(cpy * K, K)], rows.at[b], isem[b]
                ).wait()
                pltpu.async_copy(rows.at[b], out_hbm.at[cid].at[pl.ds(cpy * K, K)], gsem[b])

        for k in range(MAXD):
            b = k % NBUF
            cpy = sid + k * NS
            cpn = sid + (k + NBUF) * NS

            @pl.when((cpy < NROW_CHUNKS) & (cpn >= NROW_CHUNKS))
            def _drw():
                pltpu.make_async_copy(
                    rows.at[b], out_hbm.at[cid].at[pl.ds(0, K)], gsem[b]
                ).wait()

    return seg_sum


def _seg_sum_128(x, ei):
    return _make_seg_sum(D, D)(x, ei)


def _score_body(x_ref, p_ref, batch_ref, att_ref,
                score_ref, maskf_ref, attn_ref, ratio_ref):
    xb = x_ref[...]
    raw = jnp.sum(xb * p_ref[...], axis=1, keepdims=True)
    b = batch_ref[...]
    gi = lax.broadcasted_iota(jnp.int32, (N, G), 1)
    M = b == gi
    Mf = M.astype(jnp.float32)
    NEG = jnp.float32(-1e30)

    smax_g = jnp.max(jnp.where(M, raw, NEG), axis=0, keepdims=True)
    smax_g = jnp.where(smax_g > jnp.float32(-1e29), smax_g, 0.0)
    smax_node = jnp.sum(Mf * smax_g, axis=1, keepdims=True)
    ex = jnp.exp(raw - smax_node)
    ssum_g = jnp.sum(Mf * ex, axis=0, keepdims=True)
    ssum_node = jnp.sum(Mf * ssum_g, axis=1, keepdims=True)
    score = ex / jnp.maximum(ssum_node, 1e-16)

    scmax_g = jnp.max(jnp.where(M, score, NEG), axis=0, keepdims=True)
    scmax_g = jnp.where(scmax_g > jnp.float32(-1e29), scmax_g, 0.0)
    scmax_node = jnp.sum(Mf * scmax_g, axis=1, keepdims=True)
    thresh = jnp.minimum(scmax_node - 1e-7, MIN_SCORE_C)
    mask = score > thresh
    maskf = mask.astype(jnp.float32)

    n2 = jnp.sum(maskf)
    ratio_ref[...] = jnp.reshape(n2 * jnp.float32(1.0 / N), (1, 1))

    tgt = att_ref[...]
    kl = tgt * (jnp.log(jnp.maximum(tgt, 1e-30)) - jnp.log(score + 1e-14))
    kl = jnp.where(mask & (tgt > 0), kl, 0.0)
    counts = jnp.sum(Mf * maskf, axis=0, keepdims=True)
    attn_ref[...] = jnp.sum(Mf * kl, axis=0, keepdims=True) / jnp.maximum(counts, 1.0)

    score_ref[...] = score
    maskf_ref[...] = maskf


def _score_call(x, p, batch2, att2):
    return pl.pallas_call(
        _score_body,
        out_shape=[
            jax.ShapeDtypeStruct((N, 1), jnp.float32),
            jax.ShapeDtypeStruct((N, 1), jnp.float32),
            jax.ShapeDtypeStruct((1, G), jnp.float32),
            jax.ShapeDtypeStruct((1, 1), jnp.float32),
        ],
    )(x, p, batch2, att2)


RB = 2000
NBLK = N // RB


def _mlp1_body(a0_ref, a1_ref, x_ref, w1_ref, b1_ref, w2_ref, b2_ref,
               score_ref, maskf_ref, out_ref):
    inp = a0_ref[0] + a1_ref[0] + x_ref[...]
    t = jnp.dot(inp, w1_ref[...], preferred_element_type=jnp.float32) + b1_ref[...]
    t = jnp.maximum(t, 0.0)
    h = jnp.dot(t, w2_ref[...], preferred_element_type=jnp.float32) + b2_ref[...]
    h = jnp.maximum(h, 0.0)
    hx0 = h * score_ref[...] * maskf_ref[...]
    out_ref[...] = jnp.concatenate([hx0, jnp.zeros_like(hx0)], axis=1)


def _mlp1_call(agg, x, W1, b1, W2, b2, score, maskf):
    blk = lambda r, c: pl.BlockSpec((r, c), lambda i: (i, 0))
    full = lambda r, c: pl.BlockSpec((r, c), lambda i: (0, 0))
    part = lambda s: pl.BlockSpec((1, RB, D), lambda i, s=s: (s, i, 0))
    return pl.pallas_call(
        _mlp1_body,
        grid=(NBLK,),
        in_specs=[
            part(0), part(1), blk(RB, D),
            full(D, 256), full(1, 256), full(256, 64), full(1, 64),
            blk(RB, 1), blk(RB, 1),
        ],
        out_specs=blk(RB, 2 * 64),
        out_shape=jax.ShapeDtypeStruct((N, 2 * 64), jnp.float32),
    )(agg, agg, x, W1, b1.reshape(1, 256), W2, b2.reshape(1, 64), score, maskf)


def _mlp2_body(a0_ref, a1_ref, hx0_ref, w3_ref, b3_ref, w4_ref, b4_ref,
               w5_ref, b5_ref, maskf_ref, batch_ref, out_ref, g_scr):
    i = pl.program_id(0)
    inp = a0_ref[0] + a1_ref[0] + hx0_ref[...]
    t = jnp.dot(inp, w3_ref[...], preferred_element_type=jnp.float32) + b3_ref[...]
    t = jnp.maximum(t, 0.0)
    h2 = jnp.dot(t, w4_ref[...], preferred_element_type=jnp.float32) + b4_ref[...]
    h2 = jnp.maximum(h2, 0.0)
    h2m = h2 * maskf_ref[...]
    b = batch_ref[...]
    gi = lax.broadcasted_iota(jnp.int32, (RB, G), 1)
    Mf = (b == gi).astype(jnp.float32)
    partial = lax.dot_general(Mf, h2m, (((0,), (0,)), ((), ())),
                              preferred_element_type=jnp.float32)

    @pl.when(i == 0)
    def _():
        g_scr[...] = partial

    @pl.when(i > 0)
    def _():
        g_scr[...] = g_scr[...] + partial

    @pl.when(i == NBLK - 1)
    def _():
        out_ref[...] = (
            jnp.dot(g_scr[...], w5_ref[...], preferred_element_type=jnp.float32)
            + b5_ref[...]
        )


def _mlp2_call(agg, hx0, W3, b3, W4, b4, W5, b5, maskf, batch2):
    blk = lambda r, c: pl.BlockSpec((r, c), lambda i: (i, 0))
    full = lambda r, c: pl.BlockSpec((r, c), lambda i: (0, 0))
    part = lambda s: pl.BlockSpec((1, RB, D), lambda i, s=s: (s, i, 0))
    return pl.pallas_call(
        _mlp2_body,
        grid=(NBLK,),
        in_specs=[
            part(0), part(1), blk(RB, D),
            full(D, 256), full(1, 256), full(256, 64), full(1, 64),
            full(64, 1), full(1, 1),
            blk(RB, 1), blk(RB, 1),
        ],
        out_specs=full(G, 1),
        out_shape=jax.ShapeDtypeStruct((G, 1), jnp.float32),
        scratch_shapes=[pltpu.VMEM((G, 64), jnp.float32)],
    )(agg, agg, hx0, W3, b3.reshape(1, 256), W4, b4.reshape(1, 64),
      W5, b5.reshape(1, 1), maskf, batch2)


def kernel(x, edge_index, batch, node_attention, W1, b1, W2, b2, p, W3, b3, W4, b4, W5, b5):
    batch2 = batch.reshape(N, 1)
    att2 = node_attention.reshape(N, 1)

    ei_flat = edge_index.reshape(2 * E)
    agg1 = _seg_sum_128(x, ei_flat)

    score, maskf, attn, ratio = _score_call(x, p.reshape(1, D), batch2, att2)

    hx0 = _mlp1_call(agg1, x, W1, b1, W2, b2, score, maskf)

    agg2 = _seg_sum_128(hx0, ei_flat)

    W3p = jnp.concatenate([W3, jnp.zeros((64, 256), jnp.float32)], axis=0)
    out = _mlp2_call(agg2, hx0, W3p, b3, W4, b4, W5, b5, maskf, batch2)

    attn_loss = attn.reshape(G)
    ratio_s = ratio[0, 0]
    return out, attn_loss, ratio_s


# speedup vs baseline: 1.0235x; 1.0078x over previous
import functools

import jax
import jax.numpy as jnp
from jax import lax
from jax.experimental import pallas as pl
from jax.experimental.pallas import tpu as pltpu
from jax.experimental.pallas import tpu_sc as plsc

N = 10000
E = 320000
D = 128
G = 64
MIN_SCORE_C = 0.05

NC = 2
NS = 16
NW = NC * NS
E_PER_W = E // NW
K = 80
N_CHUNKS = E_PER_W // K
NROW_CHUNKS = N // K
NBUF = 4
NROUND = -(-N_CHUNKS // NBUF)


@functools.cache
def _make_seg_sum(d_feat, d_acc):
    mesh = plsc.VectorSubcoreMesh(
        core_axis_name="c", subcore_axis_name="s", num_cores=NC, num_subcores=NS
    )

    @functools.partial(
        pl.kernel,
        mesh=mesh,
        out_type=jax.ShapeDtypeStruct((NC, N, d_acc), jnp.float32),
        scratch_types=[
            pltpu.VMEM((NBUF, 2, K), jnp.int32),
            pltpu.VMEM((NBUF, K, d_feat), jnp.float32),
            pltpu.VMEM_SHARED((N, d_acc), jnp.float32),
        ]
        + [pltpu.SemaphoreType.DMA] * (3 * NBUF),
    )
    def seg_sum(x_hbm, ei_hbm, out_hbm, idx, rows, acc, *sems):
        isem = sems[:NBUF]
        gsem = sems[NBUF : 2 * NBUF]
        ssem = sems[2 * NBUF :]
        cid = lax.axis_index("c")
        sid = lax.axis_index("s")
        wid = sid * NC + cid

        def idx_copy(b, c, make_only=False):
            f = pltpu.make_async_copy if make_only else pltpu.async_copy
            off = wid * E_PER_W + c * K
            d0 = f(ei_hbm.at[pl.ds(off, K)], idx.at[b, 0], isem[b])
            d1 = f(ei_hbm.at[pl.ds(E + off, K)], idx.at[b, 1], isem[b])
            return (d0, d1)

        def gather_copy(b, c, make_only=False):
            f = pltpu.make_async_copy if make_only else pltpu.async_copy
            return f(x_hbm.at[idx.at[b, 0]], rows.at[b], gsem[b])

        def scatter_copy(b, c):
            pltpu.async_copy(
                rows.at[b], acc.at[idx.at[b, 1]], ssem[b], add=True
            )

        def scatter_wait(b):
            pltpu.make_async_copy(
                rows.at[b], acc.at[pl.ds(0, K)], ssem[b]
            ).wait()

        for b in range(NBUF):
            idx_copy(b, b)

        @pl.loop(0, K)
        def _zrow(i):
            @pl.loop(0, d_feat // 16)
            def _zlane(j):
                rows[0, i, pl.ds(j * 16, 16)] = jnp.zeros((16,), jnp.float32)

        @pl.loop(sid, NROW_CHUNKS, step=NS)
        def _zfire(cpy):
            pltpu.async_copy(rows.at[0], acc.at[pl.ds(cpy * K, K)], ssem[0])

        @pl.loop(sid, NROW_CHUNKS, step=NS)
        def _zwait(cpy):
            pltpu.make_async_copy(rows.at[0], acc.at[pl.ds(cpy * K, K)], ssem[0]).wait()

        plsc.subcore_barrier()

        for b in range(NBUF):
            for d in idx_copy(b, b, make_only=True):
                d.wait()
            gather_copy(b, b)

        @pl.loop(0, 30)
        def _round(r):
            c0 = r * NBUF
            for b in range(NBUF):
                gather_copy(b, c0 + b, make_only=True).wait()
                scatter_copy(b, c0 + b)
            for b in range(NBUF):
                scatter_wait(b)
                idx_copy(b, c0 + NBUF + b)
            for b in range(NBUF):
                for d in idx_copy(b, 0, make_only=True):
                    d.wait()
                gather_copy(b, c0 + NBUF + b)

        for b in range(NBUF):
            gather_copy(b, 120 + b, make_only=True).wait()
            scatter_copy(b, 120 + b)
        for b in range(NBUF):
            scatter_wait(b)
            if b == 0:
                idx_copy(0, 124)
        for d in idx_copy(0, 124, make_only=True):
            d.wait()
        gather_copy(0, 124)
        gather_copy(0, 124, make_only=True).wait()
        scatter_copy(0, 124)
        scatter_wait(0)

        plsc.subcore_barrier()

        MAXD = -(-NROW_CHUNKS // NS)
        for k in range(MAXD):
            b = k % NBUF
            cpy = sid + k * NS

            @pl.when(cpy < NROW_CHUNKS)
            def _dr():
                if k >= NBUF:
                    pltpu.make_async_copy(
                        rows.at[b], out_hbm.at[cid].at[pl.ds(0, K)], gsem[b]
                    ).wait()
                pltpu.async_copy(acc.at[pl.ds(cpy * K, K)], rows.at[b], isem[b])
                pltpu.make_async_copy(
                    acc.at[pl.ds(cpy * K, K)], rows.at[b], isem[b]
                ).wait()
                pltpu.async_copy(rows.at[b], out_hbm.at[cid].at[pl.ds(cpy * K, K)], gsem[b])

        for k in range(MAXD):
            b = k % NBUF
            cpy = sid + k * NS
            cpn = sid + (k + NBUF) * NS

            @pl.when((cpy < NROW_CHUNKS) & (cpn >= NROW_CHUNKS))
            def _drw():
                pltpu.make_async_copy(
                    rows.at[b], out_hbm.at[cid].at[pl.ds(0, K)], gsem[b]
                ).wait()

    return seg_sum


def _seg_sum_128(x, ei):
    return _make_seg_sum(D, D)(x, ei)


def _score_body(x_ref, p_ref, batch_ref, att_ref,
                score_ref, maskf_ref, attn_ref, ratio_ref):
    xb = x_ref[...]
    raw = jnp.sum(xb * p_ref[...], axis=1, keepdims=True)
    b = batch_ref[...]
    gi = lax.broadcasted_iota(jnp.int32, (N, G), 1)
    M = b == gi
    Mf = M.astype(jnp.float32)
    NEG = jnp.float32(-1e30)

    smax_g = jnp.max(jnp.where(M, raw, NEG), axis=0, keepdims=True)
    smax_g = jnp.where(smax_g > jnp.float32(-1e29), smax_g, 0.0)
    smax_node = jnp.sum(Mf * smax_g, axis=1, keepdims=True)
    ex = jnp.exp(raw - smax_node)
    ssum_g = jnp.sum(Mf * ex, axis=0, keepdims=True)
    ssum_node = jnp.sum(Mf * ssum_g, axis=1, keepdims=True)
    score = ex / jnp.maximum(ssum_node, 1e-16)

    scmax_g = jnp.max(jnp.where(M, score, NEG), axis=0, keepdims=True)
    scmax_g = jnp.where(scmax_g > jnp.float32(-1e29), scmax_g, 0.0)
    scmax_node = jnp.sum(Mf * scmax_g, axis=1, keepdims=True)
    thresh = jnp.minimum(scmax_node - 1e-7, MIN_SCORE_C)
    mask = score > thresh
    maskf = mask.astype(jnp.float32)

    n2 = jnp.sum(maskf)
    ratio_ref[...] = jnp.reshape(n2 * jnp.float32(1.0 / N), (1, 1))

    tgt = att_ref[...]
    kl = tgt * (jnp.log(jnp.maximum(tgt, 1e-30)) - jnp.log(score + 1e-14))
    kl = jnp.where(mask & (tgt > 0), kl, 0.0)
    counts = jnp.sum(Mf * maskf, axis=0, keepdims=True)
    attn_ref[...] = jnp.sum(Mf * kl, axis=0, keepdims=True) / jnp.maximum(counts, 1.0)

    score_ref[...] = score
    maskf_ref[...] = maskf


def _score_call(x, p, batch2, att2):
    return pl.pallas_call(
        _score_body,
        out_shape=[
            jax.ShapeDtypeStruct((N, 1), jnp.float32),
            jax.ShapeDtypeStruct((N, 1), jnp.float32),
            jax.ShapeDtypeStruct((1, G), jnp.float32),
            jax.ShapeDtypeStruct((1, 1), jnp.float32),
        ],
    )(x, p, batch2, att2)


RB = 5000
NBLK = N // RB


def _mlp1_body(a0_ref, a1_ref, x_ref, w1_ref, b1_ref, w2_ref, b2_ref,
               score_ref, maskf_ref, out_ref):
    inp = a0_ref[0] + a1_ref[0] + x_ref[...]
    t = jnp.dot(inp, w1_ref[...], preferred_element_type=jnp.float32) + b1_ref[...]
    t = jnp.maximum(t, 0.0)
    h = jnp.dot(t, w2_ref[...], preferred_element_type=jnp.float32) + b2_ref[...]
    h = jnp.maximum(h, 0.0)
    hx0 = h * score_ref[...] * maskf_ref[...]
    out_ref[...] = jnp.concatenate([hx0, jnp.zeros_like(hx0)], axis=1)


def _mlp1_call(agg, x, W1, b1, W2, b2, score, maskf):
    blk = lambda r, c: pl.BlockSpec((r, c), lambda i: (i, 0))
    full = lambda r, c: pl.BlockSpec((r, c), lambda i: (0, 0))
    part = lambda s: pl.BlockSpec((1, RB, D), lambda i, s=s: (s, i, 0))
    return pl.pallas_call(
        _mlp1_body,
        grid=(NBLK,),
        in_specs=[
            part(0), part(1), blk(RB, D),
            full(D, 256), full(1, 256), full(256, 64), full(1, 64),
            blk(RB, 1), blk(RB, 1),
        ],
        out_specs=blk(RB, 2 * 64),
        out_shape=jax.ShapeDtypeStruct((N, 2 * 64), jnp.float32),
    )(agg, agg, x, W1, b1.reshape(1, 256), W2, b2.reshape(1, 64), score, maskf)


def _mlp2_body(a0_ref, a1_ref, hx0_ref, w3_ref, b3_ref, w4_ref, b4_ref,
               w5_ref, b5_ref, maskf_ref, batch_ref, out_ref, g_scr):
    i = pl.program_id(0)
    inp = a0_ref[0] + a1_ref[0] + hx0_ref[...]
    t = jnp.dot(inp, w3_ref[...], preferred_element_type=jnp.float32) + b3_ref[...]
    t = jnp.maximum(t, 0.0)
    h2 = jnp.dot(t, w4_ref[...], preferred_element_type=jnp.float32) + b4_ref[...]
    h2 = jnp.maximum(h2, 0.0)
    h2m = h2 * maskf_ref[...]
    b = batch_ref[...]
    gi = lax.broadcasted_iota(jnp.int32, (RB, G), 1)
    Mf = (b == gi).astype(jnp.float32)
    partial = lax.dot_general(Mf, h2m, (((0,), (0,)), ((), ())),
                              preferred_element_type=jnp.float32)

    @pl.when(i == 0)
    def _():
        g_scr[...] = partial

    @pl.when(i > 0)
    def _():
        g_scr[...] = g_scr[...] + partial

    @pl.when(i == NBLK - 1)
    def _():
        out_ref[...] = (
            jnp.dot(g_scr[...], w5_ref[...], preferred_element_type=jnp.float32)
            + b5_ref[...]
        )


def _mlp2_call(agg, hx0, W3, b3, W4, b4, W5, b5, maskf, batch2):
    blk = lambda r, c: pl.BlockSpec((r, c), lambda i: (i, 0))
    full = lambda r, c: pl.BlockSpec((r, c), lambda i: (0, 0))
    part = lambda s: pl.BlockSpec((1, RB, D), lambda i, s=s: (s, i, 0))
    return pl.pallas_call(
        _mlp2_body,
        grid=(NBLK,),
        in_specs=[
            part(0), part(1), blk(RB, D),
            full(D, 256), full(1, 256), full(256, 64), full(1, 64),
            full(64, 1), full(1, 1),
            blk(RB, 1), blk(RB, 1),
        ],
        out_specs=full(G, 1),
        out_shape=jax.ShapeDtypeStruct((G, 1), jnp.float32),
        scratch_shapes=[pltpu.VMEM((G, 64), jnp.float32)],
    )(agg, agg, hx0, W3, b3.reshape(1, 256), W4, b4.reshape(1, 64),
      W5, b5.reshape(1, 1), maskf, batch2)


def kernel(x, edge_index, batch, node_attention, W1, b1, W2, b2, p, W3, b3, W4, b4, W5, b5):
    batch2 = batch.reshape(N, 1)
    att2 = node_attention.reshape(N, 1)

    ei_flat = edge_index.reshape(2 * E)
    agg1 = _seg_sum_128(x, ei_flat)

    score, maskf, attn, ratio = _score_call(x, p.reshape(1, D), batch2, att2)

    hx0 = _mlp1_call(agg1, x, W1, b1, W2, b2, score, maskf)

    agg2 = _seg_sum_128(hx0, ei_flat)

    W3p = jnp.concatenate([W3, jnp.zeros((64, 256), jnp.float32)], axis=0)
    out = _mlp2_call(agg2, hx0, W3p, b3, W4, b4, W5, b5, maskf, batch2)

    attn_loss = attn.reshape(G)
    ratio_s = ratio[0, 0]
    return out, attn_loss, ratio_s
